# Initial kernel scaffold; baseline (speedup 1.0000x reference)
import functools

import jax
import jax.numpy as jnp
import numpy as np
from jax import lax
from jax.experimental import pallas as pl
from jax.experimental.pallas import tpu as pltpu
from jax.experimental.pallas import tpu_sc as plsc

D_MODEL = 64
D_FFN = 1024
N_HEADS = 8
N_POINTS = 4
N_LEVELS = 5
SPATIAL = [(100, 100), (50, 50), (25, 25), (13, 13), (7, 7)]
NQ = 16384
S_TOTAL = sum(h * w for h, w in SPATIAL)
S_PAD = 13344
DH = D_MODEL // N_HEADS
LP = N_LEVELS * N_POINTS
NCOL = N_HEADS * LP
SW = S_PAD * DH
KTAP = 4 * LP
CQ = 128
NCH = (NQ // 2) // CQ
QT = 512
QT2 = 512
VT = 2224
N_TILES = 32

_lsi = np.cumsum([0] + [h * w for h, w in SPATIAL])[:-1].astype(np.float32)
_col_lvl = np.tile(np.repeat(np.arange(N_LEVELS), N_POINTS), N_HEADS)
_CST = np.zeros((8, NCOL), np.float32)
_CST[0] = np.array([w for h, w in SPATIAL], np.float32)[_col_lvl]
_CST[1] = np.array([h for h, w in SPATIAL], np.float32)[_col_lvl]
_CST[2] = 1.0 / _CST[0]
_CST[3] = 1.0 / _CST[1]
_CST[4] = _lsi[_col_lvl]


def _val_body(dv_ref, wv_ref, bv_ref, out_ref):
    out_ref[0] = (
        jnp.dot(dv_ref[0], wv_ref[...], precision=lax.Precision.HIGHEST)
        + bv_ref[...]
    )


def _run_val(dv, Wv, bv2):
    Bb = dv.shape[0]
    return pl.pallas_call(
        _val_body,
        grid=(Bb, S_PAD // VT),
        in_specs=[
            pl.BlockSpec((1, VT, D_MODEL), lambda b, i: (b, i, 0)),
            pl.BlockSpec((D_MODEL, D_MODEL), lambda b, i: (0, 0)),
            pl.BlockSpec((1, D_MODEL), lambda b, i: (0, 0)),
        ],
        out_specs=pl.BlockSpec((1, VT, D_MODEL), lambda b, i: (b, i, 0)),
        out_shape=jax.ShapeDtypeStruct((Bb, S_PAD, D_MODEL), jnp.float32),
    )(dv, Wv, bv2)


def _prep_body(q_ref, rp_ref, wox_ref, woy_ref, box_ref, boy_ref, wa_ref,
               ba_ref, cst_ref, idx_ref, cf_ref):
    hp = lax.Precision.HIGHEST
    q = q_ref[0]
    offx = jnp.dot(q, wox_ref[...], precision=hp) + box_ref[...]
    offy = jnp.dot(q, woy_ref[...], precision=hp) + boy_ref[...]
    a = jnp.dot(q, wa_ref[...], precision=hp) + ba_ref[...]
    parts = []
    for h in range(N_HEADS):
        sl = a[:, h * LP:(h + 1) * LP]
        m = jnp.max(sl, axis=1, keepdims=True)
        e = jnp.exp(sl - m)
        parts.append(e / jnp.sum(e, axis=1, keepdims=True))
    aw = jnp.concatenate(parts, axis=1)
    cst = cst_ref[...]
    wl = cst[0][None, :]
    hl = cst[1][None, :]
    invw = cst[2][None, :]
    invh = cst[3][None, :]
    start = cst[4][None, :]
    rp = rp_ref[0]
    px = rp[:, 0:1]
    py = rp[:, 1:2]
    x = (px + offx * invw) * wl - 0.5
    y = (py + offy * invh) * hl - 0.5
    x0 = jnp.floor(x)
    y0 = jnp.floor(y)
    lw = x - x0
    lh = y - y0
    idx_parts = []
    cf_parts = []
    for dxi, dyi, wgt in ((0, 0, (1 - lw) * (1 - lh)),
                          (1, 0, lw * (1 - lh)),
                          (0, 1, (1 - lw) * lh),
                          (1, 1, lw * lh)):
        xi = x0 + dxi
        yi = y0 + dyi
        valid = ((xi >= 0) & (xi < wl) & (yi >= 0) & (yi < hl))
        xc = jnp.clip(xi, 0.0, wl - 1.0)
        yc = jnp.clip(yi, 0.0, hl - 1.0)
        idxf = (yc * wl + xc + start) * float(DH)
        idx_parts.append(idxf.astype(jnp.int32))
        cf_parts.append(wgt * valid.astype(jnp.float32) * aw)
    idx_ref[0] = jnp.concatenate(idx_parts, axis=1)
    cf_ref[0] = jnp.concatenate(cf_parts, axis=1)


def _run_prep(q_feat, rp, wox, woy, box, boy, wa, ba2, cst):
    Bb = q_feat.shape[0]
    rep = lambda b, i: (0, 0)
    return pl.pallas_call(
        _prep_body,
        grid=(Bb, NQ // QT),
        in_specs=[
            pl.BlockSpec((1, QT, D_MODEL), lambda b, i: (b, i, 0)),
            pl.BlockSpec((1, QT, 2), lambda b, i: (b, i, 0)),
            pl.BlockSpec((D_MODEL, NCOL), rep),
            pl.BlockSpec((D_MODEL, NCOL), rep),
            pl.BlockSpec((1, NCOL), rep),
            pl.BlockSpec((1, NCOL), rep),
            pl.BlockSpec((D_MODEL, NCOL), rep),
            pl.BlockSpec((1, NCOL), rep),
            pl.BlockSpec((8, NCOL), rep),
        ],
        out_specs=[
            pl.BlockSpec((1, QT, 4 * NCOL), lambda b, i: (b, i, 0)),
            pl.BlockSpec((1, QT, 4 * NCOL), lambda b, i: (b, i, 0)),
        ],
        out_shape=[
            jax.ShapeDtypeStruct((Bb, NQ, 4 * NCOL), jnp.int32),
            jax.ShapeDtypeStruct((Bb, NQ, 4 * NCOL), jnp.float32),
        ],
    )(q_feat, rp, wox, woy, box, boy, wa, ba2, cst)


def _sc_body(value_hbm, idx_hbm, cf_hbm, out_hbm, table_v, idx_v, cf_v, out_v):
    w = lax.axis_index("s") * 2 + lax.axis_index("c")
    bh = w // 2
    pltpu.sync_copy(value_hbm.at[bh], table_v)

    def chunk_body(ch, carry):
        pltpu.sync_copy(idx_hbm.at[w, ch], idx_v)
        pltpu.sync_copy(cf_hbm.at[w, ch], cf_v)

        def group_body(g, carry2):
            base = pl.multiple_of(g * 16, 16)

            def tap_block(kb, accs):
                accs = list(accs)
                for t in range(16):
                    off = pl.multiple_of((kb * 16 + t) * CQ + base, 16)
                    iv = idx_v[pl.ds(off, 16)]
                    cv = cf_v[pl.ds(off, 16)]
                    for c in range(DH):
                        gv = plsc.load_gather(table_v, [iv + c])
                        accs[c] = accs[c] + cv * gv
                return tuple(accs)

            zero = jnp.zeros((16,), jnp.float32)
            accs = lax.fori_loop(0, KTAP // 16, tap_block, (zero,) * DH)
            for c in range(DH):
                out_v[pl.ds(c * CQ + base, 16)] = accs[c]
            return carry2

        lax.fori_loop(0, CQ // 16, group_body, 0)
        pltpu.sync_copy(out_v, out_hbm.at[w, ch])
        return carry

    lax.fori_loop(0, NCH, chunk_body, 0)


def _run_sc(value_t, idx_t, cf_t):
    mesh = plsc.VectorSubcoreMesh(core_axis_name="c", subcore_axis_name="s")
    f = functools.partial(
        pl.kernel,
        out_type=jax.ShapeDtypeStruct((N_TILES, NCH, DH * CQ), jnp.float32),
        mesh=mesh,
        scratch_types=[
            pltpu.VMEM((SW,), jnp.float32),
            pltpu.VMEM((KTAP * CQ,), jnp.int32),
            pltpu.VMEM((KTAP * CQ,), jnp.float32),
            pltpu.VMEM((DH * CQ,), jnp.float32),
        ],
    )(_sc_body)
    return f(value_t, idx_t, cf_t)


def _post_body(ao_ref, q_ref, wo_ref, bo_ref, g1_ref, be1_ref, w1_ref,
               bb1_ref, w2_ref, bb2_ref, g2_ref, be2_ref, out_ref):
    hp = lax.Precision.HIGHEST
    src2 = jnp.dot(ao_ref[0], wo_ref[...], precision=hp) + bo_ref[...]
    x = q_ref[0] + src2
    mu = jnp.mean(x, axis=1, keepdims=True)
    var = jnp.mean((x - mu) ** 2, axis=1, keepdims=True)
    x1 = (x - mu) * lax.rsqrt(var + 1e-5) * g1_ref[...] + be1_ref[...]
    hdn = jnp.maximum(jnp.dot(x1, w1_ref[...], precision=hp) + bb1_ref[...], 0.0)
    ff = jnp.dot(hdn, w2_ref[...], precision=hp) + bb2_ref[...]
    x2 = x1 + ff
    mu2 = jnp.mean(x2, axis=1, keepdims=True)
    var2 = jnp.mean((x2 - mu2) ** 2, axis=1, keepdims=True)
    out_ref[0] = (x2 - mu2) * lax.rsqrt(var2 + 1e-5) * g2_ref[...] + be2_ref[...]


def _run_post(ao, q_feat, Wo, bo2, g1, be1, W1, bb1, W2, bb2, g2, be2):
    Bb = q_feat.shape[0]
    rep = lambda b, i: (0, 0)
    return pl.pallas_call(
        _post_body,
        grid=(Bb, NQ // QT2),
        in_specs=[
            pl.BlockSpec((1, QT2, D_MODEL), lambda b, i: (b, i, 0)),
            pl.BlockSpec((1, QT2, D_MODEL), lambda b, i: (b, i, 0)),
            pl.BlockSpec((D_MODEL, D_MODEL), rep),
            pl.BlockSpec((1, D_MODEL), rep),
            pl.BlockSpec((1, D_MODEL), rep),
            pl.BlockSpec((1, D_MODEL), rep),
            pl.BlockSpec((D_MODEL, D_FFN), rep),
            pl.BlockSpec((1, D_FFN), rep),
            pl.BlockSpec((D_FFN, D_MODEL), rep),
            pl.BlockSpec((1, D_MODEL), rep),
            pl.BlockSpec((1, D_MODEL), rep),
            pl.BlockSpec((1, D_MODEL), rep),
        ],
        out_specs=pl.BlockSpec((1, QT2, D_MODEL), lambda b, i: (b, i, 0)),
        out_shape=jax.ShapeDtypeStruct((Bb, NQ, D_MODEL), jnp.float32),
    )(ao, q_feat, Wo, bo2, g1, be1, W1, bb1, W2, bb2, g2, be2)


def kernel(q_feat, dense_voxel_flatten, reference_points, spatial_shapes,
           level_start_index, Wv, bv, Woff, boff, Wa, ba, Wo, bo, ln1_g,
           ln1_b, W1, b1, W2, b2, ln2_g, ln2_b):
    Bb = q_feat.shape[0]
    assert Bb == 2 and q_feat.shape[1] == NQ

    dv = jnp.pad(dense_voxel_flatten, ((0, 0), (0, S_PAD - S_TOTAL), (0, 0)))
    val = _run_val(dv, Wv, bv.reshape(1, -1))
    value_t = (val.reshape(Bb, S_PAD, N_HEADS, DH)
               .transpose(0, 2, 1, 3).reshape(Bb * N_HEADS, SW))

    wox = Woff[:, 0::2]
    woy = Woff[:, 1::2]
    box = boff[0::2].reshape(1, -1)
    boy = boff[1::2].reshape(1, -1)
    idxp, cfp = _run_prep(q_feat, reference_points, wox, woy, box, boy,
                          Wa, ba.reshape(1, -1), jnp.asarray(_CST))

    def to_tiles(arr):
        return (arr.reshape(Bb, 2, NCH, CQ, 4, N_HEADS, LP)
                .transpose(0, 5, 1, 2, 4, 6, 3)
                .reshape(N_TILES, NCH, KTAP * CQ))
    idx_t = to_tiles(idxp)
    cf_t = to_tiles(cfp)

    out_sc = _run_sc(value_t, idx_t, cf_t)
    ao = (out_sc.reshape(Bb, N_HEADS, 2, NCH, DH, CQ)
          .transpose(0, 2, 3, 5, 1, 4).reshape(Bb, NQ, D_MODEL))

    return _run_post(ao, q_feat, Wo, bo.reshape(1, -1),
                     ln1_g.reshape(1, -1), ln1_b.reshape(1, -1),
                     W1, b1.reshape(1, -1), W2, b2.reshape(1, -1),
                     ln2_g.reshape(1, -1), ln2_b.reshape(1, -1))


# speedup vs baseline: 1087.3293x; 1087.3293x over previous
import functools

import jax
import jax.numpy as jnp
import numpy as np
from jax import lax
from jax.experimental import pallas as pl
from jax.experimental.pallas import tpu as pltpu
from jax.experimental.pallas import tpu_sc as plsc

D_MODEL = 64
D_FFN = 1024
N_HEADS = 8
N_POINTS = 4
N_LEVELS = 5
SPATIAL = [(100, 100), (50, 50), (25, 25), (13, 13), (7, 7)]
NQ = 16384
S_TOTAL = sum(h * w for h, w in SPATIAL)
S_PAD = 13344
DH = D_MODEL // N_HEADS
LP = N_LEVELS * N_POINTS
NCOL = N_HEADS * LP
SW = S_PAD * DH
KTAP = 4 * LP
CQ = 128
NCH = (NQ // 2) // CQ
QT = 512
QT2 = 512
VT = 2224
N_TILES = 32

_lsi = np.cumsum([0] + [h * w for h, w in SPATIAL])[:-1].astype(np.float32)
_col_lvl = np.tile(np.repeat(np.arange(N_LEVELS), N_POINTS), N_HEADS)
_CST = np.zeros((8, NCOL), np.float32)
_CST[0] = np.array([w for h, w in SPATIAL], np.float32)[_col_lvl]
_CST[1] = np.array([h for h, w in SPATIAL], np.float32)[_col_lvl]
_CST[2] = 1.0 / _CST[0]
_CST[3] = 1.0 / _CST[1]
_CST[4] = _lsi[_col_lvl]


def _val_body(dv_ref, wv_ref, bv_ref, out_ref):
    out_ref[0] = (
        jnp.dot(dv_ref[0], wv_ref[...], precision=lax.Precision.HIGHEST)
        + bv_ref[...]
    )


def _run_val(dv, Wv, bv2):
    Bb = dv.shape[0]
    return pl.pallas_call(
        _val_body,
        grid=(Bb, S_PAD // VT),
        in_specs=[
            pl.BlockSpec((1, VT, D_MODEL), lambda b, i: (b, i, 0)),
            pl.BlockSpec((D_MODEL, D_MODEL), lambda b, i: (0, 0)),
            pl.BlockSpec((1, D_MODEL), lambda b, i: (0, 0)),
        ],
        out_specs=pl.BlockSpec((1, VT, D_MODEL), lambda b, i: (b, i, 0)),
        out_shape=jax.ShapeDtypeStruct((Bb, S_PAD, D_MODEL), jnp.float32),
    )(dv, Wv, bv2)


def _prep_body(q_ref, rp_ref, wox_ref, woy_ref, box_ref, boy_ref, wa_ref,
               ba_ref, cst_ref, idx_ref, cf_ref):
    hp = lax.Precision.HIGHEST
    q = q_ref[0]
    offx = jnp.dot(q, wox_ref[...], precision=hp) + box_ref[...]
    offy = jnp.dot(q, woy_ref[...], precision=hp) + boy_ref[...]
    a = jnp.dot(q, wa_ref[...], precision=hp) + ba_ref[...]
    parts = []
    for h in range(N_HEADS):
        sl = a[:, h * LP:(h + 1) * LP]
        m = jnp.max(sl, axis=1, keepdims=True)
        e = jnp.exp(sl - m)
        parts.append(e / jnp.sum(e, axis=1, keepdims=True))
    aw = jnp.concatenate(parts, axis=1)
    cst = cst_ref[...]
    wl = cst[0][None, :]
    hl = cst[1][None, :]
    invw = cst[2][None, :]
    invh = cst[3][None, :]
    start = cst[4][None, :]
    rp = rp_ref[0]
    px = rp[:, 0:1]
    py = rp[:, 1:2]
    x = (px + offx * invw) * wl - 0.5
    y = (py + offy * invh) * hl - 0.5
    x0 = jnp.floor(x)
    y0 = jnp.floor(y)
    lw = x - x0
    lh = y - y0
    idx_parts = []
    cf_parts = []
    for dxi, dyi, wgt in ((0, 0, (1 - lw) * (1 - lh)),
                          (1, 0, lw * (1 - lh)),
                          (0, 1, (1 - lw) * lh),
                          (1, 1, lw * lh)):
        xi = x0 + dxi
        yi = y0 + dyi
        valid = ((xi >= 0) & (xi < wl) & (yi >= 0) & (yi < hl))
        xc = jnp.clip(xi, 0.0, wl - 1.0)
        yc = jnp.clip(yi, 0.0, hl - 1.0)
        idxf = (yc * wl + xc + start) * float(DH)
        idx_parts.append(idxf.astype(jnp.int32))
        cf_parts.append(wgt * valid.astype(jnp.float32) * aw)
    idx_ref[0] = jnp.concatenate(idx_parts, axis=1)
    cf_ref[0] = jnp.concatenate(cf_parts, axis=1)


def _run_prep(q_feat, rp, wox, woy, box, boy, wa, ba2, cst):
    Bb = q_feat.shape[0]
    rep = lambda b, i: (0, 0)
    return pl.pallas_call(
        _prep_body,
        grid=(Bb, NQ // QT),
        in_specs=[
            pl.BlockSpec((1, QT, D_MODEL), lambda b, i: (b, i, 0)),
            pl.BlockSpec((1, QT, 2), lambda b, i: (b, i, 0)),
            pl.BlockSpec((D_MODEL, NCOL), rep),
            pl.BlockSpec((D_MODEL, NCOL), rep),
            pl.BlockSpec((1, NCOL), rep),
            pl.BlockSpec((1, NCOL), rep),
            pl.BlockSpec((D_MODEL, NCOL), rep),
            pl.BlockSpec((1, NCOL), rep),
            pl.BlockSpec((8, NCOL), rep),
        ],
        out_specs=[
            pl.BlockSpec((1, QT, 4 * NCOL), lambda b, i: (b, i, 0)),
            pl.BlockSpec((1, QT, 4 * NCOL), lambda b, i: (b, i, 0)),
        ],
        out_shape=[
            jax.ShapeDtypeStruct((Bb, NQ, 4 * NCOL), jnp.int32),
            jax.ShapeDtypeStruct((Bb, NQ, 4 * NCOL), jnp.float32),
        ],
    )(q_feat, rp, wox, woy, box, boy, wa, ba2, cst)


def _sc_body(value_hbm, idx_hbm, cf_hbm, out_hbm, table_v, idx_v, cf_v, out_v):
    w = lax.axis_index("s") * 2 + lax.axis_index("c")
    bh = w // 2
    pltpu.sync_copy(value_hbm.at[bh], table_v)

    def chunk_body(ch, carry):
        pltpu.sync_copy(idx_hbm.at[w, ch], idx_v)
        pltpu.sync_copy(cf_hbm.at[w, ch], cf_v)

        def group_body(g, carry2):
            base = pl.multiple_of(g * 16, 16)

            def tap_block(kb, accs):
                accs = list(accs)
                for t in range(16):
                    off = pl.multiple_of((kb * 16 + t) * CQ + base, 16)
                    iv = idx_v[pl.ds(off, 16)]
                    cv = cf_v[pl.ds(off, 16)]
                    for c in range(DH):
                        gv = plsc.load_gather(table_v, [iv + c])
                        accs[c] = accs[c] + cv * gv
                return tuple(accs)

            zero = jnp.zeros((16,), jnp.float32)
            accs = lax.fori_loop(0, KTAP // 16, tap_block, (zero,) * DH)
            for c in range(DH):
                out_v[pl.ds(c * CQ + base, 16)] = accs[c]
            return carry2

        lax.fori_loop(0, CQ // 16, group_body, 0)
        pltpu.sync_copy(out_v, out_hbm.at[w, ch])
        return carry

    lax.fori_loop(0, NCH, chunk_body, 0)


def _run_sc(value_t, idx_t, cf_t):
    mesh = plsc.VectorSubcoreMesh(core_axis_name="c", subcore_axis_name="s")
    f = functools.partial(
        pl.kernel,
        out_type=jax.ShapeDtypeStruct((N_TILES, NCH, DH * CQ), jnp.float32),
        mesh=mesh,
        compiler_params=pltpu.CompilerParams(needs_layout_passes=False),
        scratch_types=[
            pltpu.VMEM((SW,), jnp.float32),
            pltpu.VMEM((KTAP * CQ,), jnp.int32),
            pltpu.VMEM((KTAP * CQ,), jnp.float32),
            pltpu.VMEM((DH * CQ,), jnp.float32),
        ],
    )(_sc_body)
    return f(value_t, idx_t, cf_t)


def _post_body(ao_ref, q_ref, wo_ref, bo_ref, g1_ref, be1_ref, w1_ref,
               bb1_ref, w2_ref, bb2_ref, g2_ref, be2_ref, out_ref):
    hp = lax.Precision.HIGHEST
    src2 = jnp.dot(ao_ref[0], wo_ref[...], precision=hp) + bo_ref[...]
    x = q_ref[0] + src2
    mu = jnp.mean(x, axis=1, keepdims=True)
    var = jnp.mean((x - mu) ** 2, axis=1, keepdims=True)
    x1 = (x - mu) * lax.rsqrt(var + 1e-5) * g1_ref[...] + be1_ref[...]
    hdn = jnp.maximum(jnp.dot(x1, w1_ref[...], precision=hp) + bb1_ref[...], 0.0)
    ff = jnp.dot(hdn, w2_ref[...], precision=hp) + bb2_ref[...]
    x2 = x1 + ff
    mu2 = jnp.mean(x2, axis=1, keepdims=True)
    var2 = jnp.mean((x2 - mu2) ** 2, axis=1, keepdims=True)
    out_ref[0] = (x2 - mu2) * lax.rsqrt(var2 + 1e-5) * g2_ref[...] + be2_ref[...]


def _run_post(ao, q_feat, Wo, bo2, g1, be1, W1, bb1, W2, bb2, g2, be2):
    Bb = q_feat.shape[0]
    rep = lambda b, i: (0, 0)
    return pl.pallas_call(
        _post_body,
        grid=(Bb, NQ // QT2),
        in_specs=[
            pl.BlockSpec((1, QT2, D_MODEL), lambda b, i: (b, i, 0)),
            pl.BlockSpec((1, QT2, D_MODEL), lambda b, i: (b, i, 0)),
            pl.BlockSpec((D_MODEL, D_MODEL), rep),
            pl.BlockSpec((1, D_MODEL), rep),
            pl.BlockSpec((1, D_MODEL), rep),
            pl.BlockSpec((1, D_MODEL), rep),
            pl.BlockSpec((D_MODEL, D_FFN), rep),
            pl.BlockSpec((1, D_FFN), rep),
            pl.BlockSpec((D_FFN, D_MODEL), rep),
            pl.BlockSpec((1, D_MODEL), rep),
            pl.BlockSpec((1, D_MODEL), rep),
            pl.BlockSpec((1, D_MODEL), rep),
        ],
        out_specs=pl.BlockSpec((1, QT2, D_MODEL), lambda b, i: (b, i, 0)),
        out_shape=jax.ShapeDtypeStruct((Bb, NQ, D_MODEL), jnp.float32),
    )(ao, q_feat, Wo, bo2, g1, be1, W1, bb1, W2, bb2, g2, be2)


def kernel(q_feat, dense_voxel_flatten, reference_points, spatial_shapes,
           level_start_index, Wv, bv, Woff, boff, Wa, ba, Wo, bo, ln1_g,
           ln1_b, W1, b1, W2, b2, ln2_g, ln2_b):
    Bb = q_feat.shape[0]
    assert Bb == 2 and q_feat.shape[1] == NQ

    dv = jnp.pad(dense_voxel_flatten, ((0, 0), (0, S_PAD - S_TOTAL), (0, 0)))
    val = _run_val(dv, Wv, bv.reshape(1, -1))
    value_t = (val.reshape(Bb, S_PAD, N_HEADS, DH)
               .transpose(0, 2, 1, 3).reshape(Bb * N_HEADS, SW))

    wox = Woff[:, 0::2]
    woy = Woff[:, 1::2]
    box = boff[0::2].reshape(1, -1)
    boy = boff[1::2].reshape(1, -1)
    idxp, cfp = _run_prep(q_feat, reference_points, wox, woy, box, boy,
                          Wa, ba.reshape(1, -1), jnp.asarray(_CST))

    def to_tiles(arr):
        return (arr.reshape(Bb, 2, NCH, CQ, 4, N_HEADS, LP)
                .transpose(0, 5, 1, 2, 4, 6, 3)
                .reshape(N_TILES, NCH, KTAP * CQ))
    idx_t = to_tiles(idxp)
    cf_t = to_tiles(cfp)

    out_sc = _run_sc(value_t, idx_t, cf_t)
    ao = (out_sc.reshape(Bb, N_HEADS, 2, NCH, DH, CQ)
          .transpose(0, 2, 3, 5, 1, 4).reshape(Bb, NQ, D_MODEL))

    return _run_post(ao, q_feat, Wo, bo.reshape(1, -1),
                     ln1_g.reshape(1, -1), ln1_b.reshape(1, -1),
                     W1, b1.reshape(1, -1), W2, b2.reshape(1, -1),
                     ln2_g.reshape(1, -1), ln2_b.reshape(1, -1))
